# Initial kernel scaffold; baseline (speedup 1.0000x reference)
import jax
import jax.numpy as jnp
from jax.experimental import pallas as pl


def kernel(x, edge_index, batch, num_graphs, num_nodes, W1, b1, W2, b2, g1, be1, W3, b3, W4, b4, g2, be2):
    raise NotImplementedError("write your pallas kernel here")


# speedup vs baseline: 4.7322x; 4.7322x over previous
import functools

import jax
import jax.numpy as jnp
from jax import lax
from jax.experimental import pallas as pl
from jax.experimental.pallas import tpu as pltpu
from jax.experimental.pallas import tpu_sc as plsc

N = 10000
E = 320000
G = 8
F_IN = 128
D1 = 32
D2 = 64
EMB = 64
BN_EPS = 1e-5

W = 128
ACC_ROWS = 10240
DUMMY = N
CHUNK = 80
NSC = 2
NSUB = 16
TILES = NSC * NSUB
EPT = E // TILES
NCHUNK = EPT // CHUNK
ZROWS = ACC_ROWS // NSUB


@functools.lru_cache(maxsize=None)
def _make_seg_sum():
    mesh = plsc.VectorSubcoreMesh(core_axis_name="c", subcore_axis_name="s",
                                  num_cores=NSC, num_subcores=NSUB)

    @functools.partial(
        pl.kernel,
        out_type=jax.ShapeDtypeStruct((NSC, ACC_ROWS, W), jnp.float32),
        mesh=mesh,
        scratch_types=[
            pltpu.VMEM((CHUNK,), jnp.int32),
            pltpu.VMEM((CHUNK,), jnp.int32),
            pltpu.VMEM((CHUNK, W), jnp.float32),
            pltpu.VMEM_SHARED((ACC_ROWS, W), jnp.float32),
            pltpu.SemaphoreType.DMA,
        ],
    )
    def seg(tab_hbm, src_hbm, dst_hbm, zer_hbm, out_hbm, src_v, dst_v, rows_v,
            acc, sem):
        c = lax.axis_index("c")
        s = lax.axis_index("s")
        tile = c * NSUB + s

        pltpu.sync_copy(zer_hbm.at[pl.ds(s * ZROWS, ZROWS)],
                        acc.at[pl.ds(s * ZROWS, ZROWS)])
        plsc.subcore_barrier()

        def body(i, carry):
            base = tile * EPT + i * CHUNK
            off = (base // 40000) * 1250
            pltpu.sync_copy(src_hbm.at[pl.ds(base, CHUNK)], src_v)
            pltpu.sync_copy(dst_hbm.at[pl.ds(base, CHUNK)], dst_v)
            for k in range(CHUNK // 16):
                sl = pl.ds(k * 16, 16)
                sv = src_v[sl]
                dv = dst_v[sl]
                ok = (sv >= off) & (dv >= off)
                dst_v[sl] = jnp.where(ok, dv, DUMMY)
            pltpu.async_copy(tab_hbm.at[src_v], rows_v, sem).wait()
            pltpu.sync_copy(rows_v, acc.at[dst_v], add=True)
            return carry

        lax.fori_loop(0, NCHUNK, body, 0)
        plsc.subcore_barrier()
        pltpu.sync_copy(acc.at[pl.ds(s * ZROWS, ZROWS)],
                        out_hbm.at[c, pl.ds(s * ZROWS, ZROWS)])

    return seg


def _mlp1_body(x_ref, p_ref, w1_ref, b1_ref, w2_ref, b2_ref, g1_ref, be1_ref,
               w3_ref, z_ref):
    h = x_ref[...] + p_ref[0, :N, :] + p_ref[1, :N, :]
    h = jnp.dot(h, w1_ref[...], preferred_element_type=jnp.float32) + b1_ref[...]
    h = jnp.maximum(h, 0.0)
    h = jnp.dot(h, w2_ref[...], preferred_element_type=jnp.float32) + b2_ref[...]
    h = jnp.maximum(h, 0.0)
    m = jnp.mean(h, axis=0)
    v = jnp.mean((h - m) ** 2, axis=0)
    h = (h - m) / jnp.sqrt(v + BN_EPS) * g1_ref[...] + be1_ref[...]
    z = jnp.dot(h, w3_ref[...], preferred_element_type=jnp.float32)
    z_ref[...] = jnp.concatenate(
        [z, jnp.zeros((N, W - D2), jnp.float32)], axis=1)


def _mlp2_body(z_ref, q_ref, b3_ref, w4_ref, b4_ref, g2_ref, be2_ref,
               batch_ref, o_ref):
    h = z_ref[:, :D2] + q_ref[0, :N, :D2] + q_ref[1, :N, :D2] + b3_ref[...]
    h = jnp.maximum(h, 0.0)
    h = jnp.dot(h, w4_ref[...], preferred_element_type=jnp.float32) + b4_ref[...]
    h = jnp.maximum(h, 0.0)
    m = jnp.mean(h, axis=0)
    v = jnp.mean((h - m) ** 2, axis=0)
    h = (h - m) / jnp.sqrt(v + BN_EPS) * g2_ref[...] + be2_ref[...]
    onehot = (batch_ref[...] ==
              lax.broadcasted_iota(jnp.int32, (1, G), 1)).astype(jnp.float32)
    sums = lax.dot_general(onehot, h, (((0,), (0,)), ((), ())),
                           preferred_element_type=jnp.float32)
    cnt = jnp.sum(onehot, axis=0)
    o_ref[...] = sums / jnp.maximum(cnt, 1.0)[:, None]


def kernel(x, edge_index, batch, num_graphs, num_nodes,
           W1, b1, W2, b2, g1, be1, W3, b3, W4, b4, g2, be2):
    src = edge_index[0]
    dst = edge_index[1]
    zer = jnp.zeros((ACC_ROWS, W), jnp.float32)

    p = _make_seg_sum()(x, src, dst, zer)

    z = pl.pallas_call(
        _mlp1_body,
        out_shape=jax.ShapeDtypeStruct((N, W), jnp.float32),
    )(x, p, W1, b1, W2, b2, g1, be1, W3)

    q = _make_seg_sum()(z, src, dst, zer)

    out = pl.pallas_call(
        _mlp2_body,
        out_shape=jax.ShapeDtypeStruct((G, EMB), jnp.float32),
    )(z, q, b3, W4, b4, g2, be2, batch.reshape(N, 1))
    return out


# speedup vs baseline: 7.4344x; 1.5710x over previous
import functools

import jax
import jax.numpy as jnp
from jax import lax
from jax.experimental import pallas as pl
from jax.experimental.pallas import tpu as pltpu
from jax.experimental.pallas import tpu_sc as plsc

N = 10000
E = 320000
G = 8
F_IN = 128
D1 = 32
D2 = 64
EMB = 64
BN_EPS = 1e-5

W = 128
ACC_ROWS = 10240
DUMMY = N
CHUNK = 80
NSC = 2
NSUB = 16
TILES = NSC * NSUB
EPT = E // TILES
NCHUNK = EPT // CHUNK
ZROWS = ACC_ROWS // NSUB


@functools.lru_cache(maxsize=None)
def _make_seg_sum():
    mesh = plsc.VectorSubcoreMesh(core_axis_name="c", subcore_axis_name="s",
                                  num_cores=NSC, num_subcores=NSUB)

    @functools.partial(
        pl.kernel,
        out_type=jax.ShapeDtypeStruct((NSC, ACC_ROWS, W), jnp.float32),
        mesh=mesh,
        scratch_types=[
            pltpu.VMEM((EPT,), jnp.int32),
            pltpu.VMEM((NCHUNK, CHUNK), jnp.int32),
            pltpu.VMEM((2, CHUNK, W), jnp.float32),
            pltpu.VMEM_SHARED((ACC_ROWS, W), jnp.float32),
            pltpu.SemaphoreType.DMA,
            pltpu.SemaphoreType.DMA,
            pltpu.SemaphoreType.DMA,
        ],
    )
    def seg(tab_hbm, src_hbm, dst_hbm, zer_hbm, out_hbm, src_v, dm_v,
            rows_v, acc, sem0, sem1, zsem):
        c = lax.axis_index("c")
        s = lax.axis_index("s")
        tile = c * NSUB + s
        base = tile * EPT

        zcp = pltpu.async_copy(zer_hbm.at[pl.ds(s * ZROWS, ZROWS)],
                               acc.at[pl.ds(s * ZROWS, ZROWS)], zsem)
        pltpu.sync_copy(src_hbm.at[pl.ds(base, EPT)], src_v)

        def dst_fire(i, carry):
            pltpu.async_copy(dst_hbm.at[pl.ds(base + i * CHUNK, CHUNK)],
                             dm_v.at[i], sem0)
            return carry

        lax.fori_loop(0, NCHUNK, dst_fire, 0)

        def dst_drain(i, carry):
            pltpu.make_async_copy(dst_hbm.at[pl.ds(base, CHUNK)],
                                  dm_v.at[0], sem0).wait()
            return carry

        lax.fori_loop(0, NCHUNK, dst_drain, 0)

        def mask_body(i, carry):
            off = ((base + i * CHUNK) // 40000) * 1250
            for k in range(CHUNK // 16):
                sl = pl.ds(k * 16, 16)
                sv = src_v[pl.ds(i * CHUNK + k * 16, 16)]
                dv = dm_v[i, sl]
                ok = (sv >= off) & (dv >= off)
                dm_v[i, sl] = jnp.where(ok, dv, DUMMY)
            return carry

        lax.fori_loop(0, NCHUNK, mask_body, 0)
        zcp.wait()
        plsc.subcore_barrier()

        sems = (sem0, sem1)

        def gather(i, b):
            return pltpu.async_copy(
                tab_hbm.at[src_v.at[pl.ds(i * CHUNK, CHUNK)]],
                rows_v.at[b], sems[b])

        gather(0, 0)
        gather(1, 1)

        def pair_body(j, carry):
            i0 = 2 * j
            pltpu.make_async_copy(tab_hbm.at[src_v.at[pl.ds(0, CHUNK)]],
                                  rows_v.at[0], sem0).wait()
            pltpu.sync_copy(rows_v.at[0], acc.at[dm_v.at[i0]], add=True)

            @pl.when(i0 + 2 < NCHUNK)
            def _():
                gather(i0 + 2, 0)

            pltpu.make_async_copy(tab_hbm.at[src_v.at[pl.ds(0, CHUNK)]],
                                  rows_v.at[1], sem1).wait()
            pltpu.sync_copy(rows_v.at[1], acc.at[dm_v.at[i0 + 1]], add=True)

            @pl.when(i0 + 3 < NCHUNK)
            def _():
                gather(i0 + 3, 1)

            return carry

        lax.fori_loop(0, NCHUNK // 2, pair_body, 0)
        pltpu.make_async_copy(tab_hbm.at[src_v.at[pl.ds(0, CHUNK)]],
                              rows_v.at[0], sem0).wait()
        pltpu.sync_copy(rows_v.at[0], acc.at[dm_v.at[NCHUNK - 1]], add=True)

        plsc.subcore_barrier()
        pltpu.sync_copy(acc.at[pl.ds(s * ZROWS, ZROWS)],
                        out_hbm.at[c, pl.ds(s * ZROWS, ZROWS)])

    return seg


def _mlp1_body(x_ref, p_ref, w1_ref, b1_ref, w2_ref, b2_ref, g1_ref, be1_ref,
               w3_ref, z_ref):
    h = x_ref[...] + p_ref[0, :N, :] + p_ref[1, :N, :]
    h = jnp.dot(h, w1_ref[...], preferred_element_type=jnp.float32) + b1_ref[...]
    h = jnp.maximum(h, 0.0)
    h = jnp.dot(h, w2_ref[...], preferred_element_type=jnp.float32) + b2_ref[...]
    h = jnp.maximum(h, 0.0)
    m = jnp.mean(h, axis=0)
    v = jnp.mean((h - m) ** 2, axis=0)
    h = (h - m) / jnp.sqrt(v + BN_EPS) * g1_ref[...] + be1_ref[...]
    z = jnp.dot(h, w3_ref[...], preferred_element_type=jnp.float32)
    z_ref[...] = jnp.concatenate(
        [z, jnp.zeros((N, W - D2), jnp.float32)], axis=1)


def _mlp2_body(z_ref, q_ref, b3_ref, w4_ref, b4_ref, g2_ref, be2_ref,
               batch_ref, o_ref):
    h = z_ref[:, :D2] + q_ref[0, :N, :D2] + q_ref[1, :N, :D2] + b3_ref[...]
    h = jnp.maximum(h, 0.0)
    h = jnp.dot(h, w4_ref[...], preferred_element_type=jnp.float32) + b4_ref[...]
    h = jnp.maximum(h, 0.0)
    m = jnp.mean(h, axis=0)
    v = jnp.mean((h - m) ** 2, axis=0)
    h = (h - m) / jnp.sqrt(v + BN_EPS) * g2_ref[...] + be2_ref[...]
    onehot = (batch_ref[...] ==
              lax.broadcasted_iota(jnp.int32, (1, G), 1)).astype(jnp.float32)
    sums = lax.dot_general(onehot, h, (((0,), (0,)), ((), ())),
                           preferred_element_type=jnp.float32)
    cnt = jnp.sum(onehot, axis=0)
    o_ref[...] = sums / jnp.maximum(cnt, 1.0)[:, None]


def kernel(x, edge_index, batch, num_graphs, num_nodes,
           W1, b1, W2, b2, g1, be1, W3, b3, W4, b4, g2, be2):
    src = edge_index[0]
    dst = edge_index[1]
    zer = jnp.zeros((ACC_ROWS, W), jnp.float32)

    p = _make_seg_sum()(x, src, dst, zer)

    z = pl.pallas_call(
        _mlp1_body,
        out_shape=jax.ShapeDtypeStruct((N, W), jnp.float32),
    )(x, p, W1, b1, W2, b2, g1, be1, W3)

    q = _make_seg_sum()(z, src, dst, zer)

    out = pl.pallas_call(
        _mlp2_body,
        out_shape=jax.ShapeDtypeStruct((G, EMB), jnp.float32),
    )(z, q, b3, W4, b4, g2, be2, batch.reshape(N, 1))
    return out
